# Initial kernel scaffold; baseline (speedup 1.0000x reference)
import jax
import jax.numpy as jnp
from jax.experimental import pallas as pl


def kernel(x, edge_index, edge_hidden, W_msg_w, W_msg_b, W_node_w, W_node_b):
    raise NotImplementedError("write your pallas kernel here")


# speedup vs baseline: 1.0565x; 1.0565x over previous
import jax
import jax.numpy as jnp
from jax.experimental import pallas as pl


def _lin_relu_kernel(a_ref, b_ref, wa_ref, wb_ref, bias_ref, out_ref):
    acc = jnp.dot(a_ref[...], wa_ref[...], preferred_element_type=jnp.float32)
    acc = acc + jnp.dot(b_ref[...], wb_ref[...],
                        preferred_element_type=jnp.float32)
    acc = acc + bias_ref[...]
    out_ref[...] = jnp.maximum(acc, 0.0)


def _lin_relu(a, b, w, bias, block_rows):
    rows = a.shape[0]
    da = a.shape[1]
    h = w.shape[0]
    wa = w[:, :da].T
    wb = w[:, da:].T
    bias2 = bias.reshape(1, h)
    grid = (rows // block_rows,)
    return pl.pallas_call(
        _lin_relu_kernel,
        grid=grid,
        in_specs=[
            pl.BlockSpec((block_rows, da), lambda i: (i, 0)),
            pl.BlockSpec((block_rows, b.shape[1]), lambda i: (i, 0)),
            pl.BlockSpec((da, h), lambda i: (0, 0)),
            pl.BlockSpec((b.shape[1], h), lambda i: (0, 0)),
            pl.BlockSpec((1, h), lambda i: (0, 0)),
        ],
        out_specs=pl.BlockSpec((block_rows, h), lambda i: (i, 0)),
        out_shape=jax.ShapeDtypeStruct((rows, h), jnp.float32),
    )(a, b, wa, wb, bias2)


def kernel(x, edge_index, edge_hidden, W_msg_w, W_msg_b, W_node_w, W_node_b):
    row = edge_index[0]
    col = edge_index[1]
    n = x.shape[0]
    e = row.shape[0]

    out_sum = jax.ops.segment_sum(edge_hidden, row, num_segments=n)
    agg_all = out_sum[col]

    keys = row * n + col
    order = jnp.argsort(keys)
    skeys = keys[order]
    seh = edge_hidden[order]
    new_run = jnp.concatenate([
        jnp.zeros((1,), dtype=jnp.int32),
        (skeys[1:] != skeys[:-1]).astype(jnp.int32),
    ])
    seg_ids = jnp.cumsum(new_run)
    run_sums = jax.ops.segment_sum(seh, seg_ids, num_segments=e)
    rev_keys = col * n + row
    pos = jnp.searchsorted(skeys, rev_keys, side='left')
    posc = jnp.clip(pos, 0, e - 1)
    match = (pos < e) & (skeys[posc] == rev_keys)
    rev_sum = jnp.where(match[:, None], run_sums[seg_ids[posc]], 0.0)
    msg_agg = agg_all - rev_sum

    messages = _lin_relu(x[row], msg_agg, W_msg_w, W_msg_b, block_rows=5000)

    node_messages = jax.ops.segment_sum(messages, col, num_segments=n)
    x_out = _lin_relu(x, node_messages, W_node_w, W_node_b, block_rows=5000)
    return (x_out, messages)


# speedup vs baseline: 1.0812x; 1.0234x over previous
import jax
import jax.numpy as jnp
from jax.experimental import pallas as pl


def _matmul_kernel(a_ref, w_ref, out_ref):
    out_ref[...] = jnp.dot(a_ref[...], w_ref[...],
                           preferred_element_type=jnp.float32)


def _edge_kernel(y_ref, agg_ref, wb_ref, bias_ref, out_ref):
    acc = y_ref[...] + jnp.dot(agg_ref[...], wb_ref[...],
                               preferred_element_type=jnp.float32)
    out_ref[...] = jnp.maximum(acc + bias_ref[...], 0.0)


def _lin_relu_kernel(a_ref, b_ref, wa_ref, wb_ref, bias_ref, out_ref):
    acc = jnp.dot(a_ref[...], wa_ref[...], preferred_element_type=jnp.float32)
    acc = acc + jnp.dot(b_ref[...], wb_ref[...],
                        preferred_element_type=jnp.float32)
    out_ref[...] = jnp.maximum(acc + bias_ref[...], 0.0)


def _matmul(a, w, block_rows):
    rows, da = a.shape
    h = w.shape[1]
    return pl.pallas_call(
        _matmul_kernel,
        grid=(rows // block_rows,),
        in_specs=[
            pl.BlockSpec((block_rows, da), lambda i: (i, 0)),
            pl.BlockSpec((da, h), lambda i: (0, 0)),
        ],
        out_specs=pl.BlockSpec((block_rows, h), lambda i: (i, 0)),
        out_shape=jax.ShapeDtypeStruct((rows, h), jnp.float32),
    )(a, w)


def _edge_stage(y_row, agg, wb, bias, block_rows):
    rows, h = y_row.shape
    return pl.pallas_call(
        _edge_kernel,
        grid=(rows // block_rows,),
        in_specs=[
            pl.BlockSpec((block_rows, h), lambda i: (i, 0)),
            pl.BlockSpec((block_rows, agg.shape[1]), lambda i: (i, 0)),
            pl.BlockSpec((agg.shape[1], h), lambda i: (0, 0)),
            pl.BlockSpec((1, h), lambda i: (0, 0)),
        ],
        out_specs=pl.BlockSpec((block_rows, h), lambda i: (i, 0)),
        out_shape=jax.ShapeDtypeStruct((rows, h), jnp.float32),
    )(y_row, agg, wb, bias.reshape(1, h))


def _lin_relu(a, b, w, bias, block_rows):
    rows = a.shape[0]
    da = a.shape[1]
    h = w.shape[0]
    wa = w[:, :da].T
    wb = w[:, da:].T
    return pl.pallas_call(
        _lin_relu_kernel,
        grid=(rows // block_rows,),
        in_specs=[
            pl.BlockSpec((block_rows, da), lambda i: (i, 0)),
            pl.BlockSpec((block_rows, b.shape[1]), lambda i: (i, 0)),
            pl.BlockSpec((da, h), lambda i: (0, 0)),
            pl.BlockSpec((b.shape[1], h), lambda i: (0, 0)),
            pl.BlockSpec((1, h), lambda i: (0, 0)),
        ],
        out_specs=pl.BlockSpec((block_rows, h), lambda i: (i, 0)),
        out_shape=jax.ShapeDtypeStruct((rows, h), jnp.float32),
    )(a, b, wa, wb, bias.reshape(1, h))


def kernel(x, edge_index, edge_hidden, W_msg_w, W_msg_b, W_node_w, W_node_b):
    row = edge_index[0]
    col = edge_index[1]
    n = x.shape[0]
    e = row.shape[0]

    out_sum = jax.ops.segment_sum(edge_hidden, row, num_segments=n)
    agg_all = out_sum[col]

    keys = row * n + col
    order = jnp.argsort(keys)
    skeys = keys[order]
    seh = edge_hidden[order]
    new_run = jnp.concatenate([
        jnp.zeros((1,), dtype=jnp.int32),
        (skeys[1:] != skeys[:-1]).astype(jnp.int32),
    ])
    seg_ids = jnp.cumsum(new_run)
    run_sums = jax.ops.segment_sum(seh, seg_ids, num_segments=e)
    rev_keys = col * n + row
    pos = jnp.searchsorted(skeys, rev_keys, side='left')
    posc = jnp.clip(pos, 0, e - 1)
    match = (pos < e) & (skeys[posc] == rev_keys)
    rev_sum = jnp.where(match[:, None], run_sums[seg_ids[posc]], 0.0)
    msg_agg = agg_all - rev_sum

    da = x.shape[1]
    y = _matmul(x, W_msg_w[:, :da].T, block_rows=5000)
    messages = _edge_stage(y[row], msg_agg, W_msg_w[:, da:].T, W_msg_b,
                           block_rows=5000)

    node_messages = jax.ops.segment_sum(messages, col, num_segments=n)
    x_out = _lin_relu(x, node_messages, W_node_w, W_node_b, block_rows=5000)
    return (x_out, messages)
